# baseline (speedup 1.0000x reference)
import functools

import jax
import jax.numpy as jnp
from jax import lax
from jax.experimental import pallas as pl
from jax.experimental.pallas import tpu as pltpu
from jax.experimental.pallas import tpu_sc as plsc

_N_NODES = 10000
_ROW = 128
_NUM_TOK = 16
_TOK = 8
_EMB_ROWS = 256
_EMB_DIM = 8
_NW = 32
_SPAN = -(-_N_NODES // _NW)
_SPANW = _SPAN * _ROW
_CODES = _SPAN * _NUM_TOK

_N_EDGES = 320000
_EDGEW = _N_EDGES * _ROW
_EW_PER_W = _EDGEW // _NW
_ZBUF = 32000
_NZ = _EW_PER_W // _ZBUF
assert _NZ * _ZBUF == _EW_PER_W
_ZRING = 8

_mesh = plsc.VectorSubcoreMesh(core_axis_name="c", subcore_axis_name="s")


@functools.partial(
    pl.kernel,
    out_type=(
        jax.ShapeDtypeStruct((_N_NODES * _ROW,), jnp.float32),
        jax.ShapeDtypeStruct((_EDGEW,), jnp.float32),
    ),
    mesh=_mesh,
    compiler_params=pltpu.CompilerParams(
        needs_layout_passes=False, use_tc_tiling_on_sc=False),
    scratch_types=[
        pltpu.VMEM((_SPANW,), jnp.int32),
        pltpu.VMEM((2048,), jnp.float32),
        pltpu.VMEM((_CODES,), jnp.int32),
        pltpu.VMEM((_SPANW,), jnp.float32),
        pltpu.VMEM((_ZBUF,), jnp.float32),
        pltpu.SemaphoreType.DMA,
    ],
)
def _node_emb(x_hbm, emb_hbm, out_hbm, edge_hbm, xv, embv, codesv, outv,
              zv, zsem):
    wid = lax.axis_index("s") * 2 + lax.axis_index("c")
    start = jnp.minimum(wid * _SPAN, _N_NODES - _SPAN)

    lanes = lax.iota(jnp.int32, 16)
    col_base = lanes * _TOK
    epat = lanes & 7
    pair_base = lanes >> 3
    zero16 = jnp.zeros((16,), jnp.float32)

    def zinit(i, carry):
        zv[pl.ds(i * 16, 16)] = zero16
        return carry

    lax.fori_loop(0, _ZBUF // 16, zinit, 0, unroll=8)

    ebase = wid * _EW_PER_W

    def zfire(i):
        return pltpu.async_copy(
            zv, edge_hbm.at[pl.ds(ebase + i * _ZBUF, _ZBUF)], zsem)

    zdescs = [zfire(i) for i in range(_ZRING)]

    pltpu.sync_copy(emb_hbm, embv)
    pltpu.sync_copy(x_hbm.at[pl.ds(start * _ROW, _SPANW)], xv)

    def pack_node(n, carry):
        nbase = col_base + n * _ROW
        codes = plsc.load_gather(xv, [nbase])
        for b in range(1, _TOK):
            plane = plsc.load_gather(xv, [nbase + b])
            codes = codes + (plane << b)
        codesv[pl.ds(n * _NUM_TOK, _NUM_TOK)] = codes
        return carry

    lax.fori_loop(0, _SPAN, pack_node, 0, unroll=8)

    def expand_node(n, carry):
        cbase = n * _NUM_TOK + pair_base
        cpairs = [plsc.load_gather(codesv, [cbase + 2 * v])
                  for v in range(_ROW // 16)]
        vals = [plsc.load_gather(embv, [(c << 3) + epat]) for c in cpairs]
        for v, val in enumerate(vals):
            outv[pl.ds(n * _ROW + v * 16, 16)] = val
        return carry

    lax.fori_loop(0, _SPAN, expand_node, 0, unroll=4)

    pltpu.sync_copy(outv, out_hbm.at[pl.ds(start * _ROW, _SPANW)])

    for i in range(_ZRING, _NZ):
        zdescs[i - _ZRING].wait()
        zdescs.append(zfire(i))
    for d in zdescs[_NZ - _ZRING:]:
        d.wait()


def kernel(x, edge_index, emb_table):
    node_flat, edge_flat = _node_emb(
        x.reshape(-1).astype(jnp.int32), emb_table.reshape(-1))
    node_vec = node_flat.reshape(_N_NODES, _ROW)
    edge_vec = edge_flat.reshape(_N_EDGES, _ROW)
    return (node_vec, edge_index, edge_vec)


# speedup vs baseline: 1.0284x; 1.0284x over previous
import functools

import jax
import jax.numpy as jnp
from jax import lax
from jax.experimental import pallas as pl
from jax.experimental.pallas import tpu as pltpu
from jax.experimental.pallas import tpu_sc as plsc

_N_NODES = 10000
_ROW = 128
_NUM_TOK = 16
_TOK = 8
_EMB_ROWS = 256
_EMB_DIM = 8
_NW = 32
_SPAN = -(-_N_NODES // _NW)
_SPANW = _SPAN * _ROW
_CODES = _SPAN * _NUM_TOK

_mesh = plsc.VectorSubcoreMesh(core_axis_name="c", subcore_axis_name="s")


@functools.partial(
    pl.kernel,
    out_type=jax.ShapeDtypeStruct((_N_NODES * _ROW,), jnp.float32),
    mesh=_mesh,
    compiler_params=pltpu.CompilerParams(
        needs_layout_passes=False, use_tc_tiling_on_sc=False),
    cost_estimate=pl.CostEstimate(
        flops=400_000_000, transcendentals=0, bytes_accessed=400_000_000),
    scratch_types=[
        pltpu.VMEM((_SPANW,), jnp.int32),
        pltpu.VMEM((2048,), jnp.float32),
        pltpu.VMEM((_CODES,), jnp.int32),
        pltpu.VMEM((_SPANW,), jnp.float32),
    ],
)
def _node_emb(x_hbm, emb_hbm, out_hbm, xv, embv, codesv, outv):
    wid = lax.axis_index("s") * 2 + lax.axis_index("c")
    start = jnp.minimum(wid * _SPAN, _N_NODES - _SPAN)

    lanes = lax.iota(jnp.int32, 16)
    col_base = lanes * _TOK
    epat = lanes & 7
    pair_base = lanes >> 3

    pltpu.sync_copy(emb_hbm, embv)
    pltpu.sync_copy(x_hbm.at[pl.ds(start * _ROW, _SPANW)], xv)

    def pack_node(n, carry):
        nbase = col_base + n * _ROW
        codes = plsc.load_gather(xv, [nbase])
        for b in range(1, _TOK):
            plane = plsc.load_gather(xv, [nbase + b])
            codes = codes + (plane << b)
        codesv[pl.ds(n * _NUM_TOK, _NUM_TOK)] = codes
        return carry

    lax.fori_loop(0, _SPAN, pack_node, 0, unroll=8)

    def expand_node(n, carry):
        cbase = n * _NUM_TOK + pair_base
        cpairs = [plsc.load_gather(codesv, [cbase + 2 * v])
                  for v in range(_ROW // 16)]
        vals = [plsc.load_gather(embv, [(c << 3) + epat]) for c in cpairs]
        for v, val in enumerate(vals):
            outv[pl.ds(n * _ROW + v * 16, 16)] = val
        return carry

    lax.fori_loop(0, _SPAN, expand_node, 0, unroll=4)

    pltpu.sync_copy(outv, out_hbm.at[pl.ds(start * _ROW, _SPANW)])


def kernel(x, edge_index, emb_table):
    node_flat = _node_emb(
        x.reshape(-1).astype(jnp.int32), emb_table.reshape(-1))
    node_vec = node_flat.reshape(_N_NODES, _ROW)
    edge_vec = jnp.zeros((edge_index.shape[-1], _ROW), dtype=jnp.float32)
    return (node_vec, edge_index, edge_vec)
